# Initial kernel scaffold; baseline (speedup 1.0000x reference)
import jax
import jax.numpy as jnp
from jax.experimental import pallas as pl


def kernel(positions, hidden_states, Wq, Wk, Wv, Wo, q_norm_w, k_norm_w, in_ln_w, post_ln_w, Wg, We_gate, We_up, We_down, Ws_gate, Ws_up, Ws_down):
    raise NotImplementedError("write your pallas kernel here")


# speedup vs baseline: 1.8212x; 1.8212x over previous
import functools

import jax
import jax.numpy as jnp
from jax.experimental import pallas as pl
from jax.experimental.pallas import tpu as pltpu

H = 768
NH = 12
NKV = 4
HD = 64
E = 8
TOPK = 2
DFF = 512
T = 2048
THETA = 1000000.0
EPS = 1e-06
REP = NH // NKV


def _bf(x):
    return x.astype(jnp.bfloat16)


def _dot(a, b):
    return jax.lax.dot_general(
        _bf(a), _bf(b), (((1,), (0,)), ((), ())),
        preferred_element_type=jnp.float32)


def _rms(x, w):
    v = jnp.mean(jnp.square(x), axis=-1, keepdims=True)
    return x * jax.lax.rsqrt(v + EPS) * w


def _qkv_kernel(pos_ref, hs_ref, wq_ref, wk_ref, wv_ref, qn_ref, kn_ref,
                ln_ref, qo_ref, ko_ref, vo_ref):
    hs = hs_ref[...]
    h = _rms(hs, ln_ref[...])
    q = _dot(h, wq_ref[...])
    k = _dot(h, wk_ref[...])
    v = _dot(h, wv_ref[...])

    pos = pos_ref[...].astype(jnp.float32)
    half = HD // 2
    exponent = jax.lax.broadcasted_iota(jnp.int32, (1, half), 1).astype(
        jnp.float32) * (2.0 / HD)
    inv_freq = jnp.exp(-jnp.log(THETA) * exponent)
    ang = pos * inv_freq
    cos = jnp.cos(ang)
    sin = jnp.sin(ang)

    qw = qn_ref[...].reshape(1, HD)
    kw = kn_ref[...].reshape(1, HD)

    def norm_rope(x, w):
        xn = _rms(x, w)
        x1 = xn[:, :half]
        x2 = xn[:, half:]
        return jnp.concatenate([x1 * cos - x2 * sin, x2 * cos + x1 * sin],
                               axis=-1)

    for hh in range(NH):
        qo_ref[hh] = norm_rope(q[:, hh * HD:(hh + 1) * HD], qw)
    for hh in range(NKV):
        ko_ref[hh] = norm_rope(k[:, hh * HD:(hh + 1) * HD], kw)
        vo_ref[hh] = v[:, hh * HD:(hh + 1) * HD]


def _attn_kernel(q_ref, k_ref, v_ref, o_ref):
    q = q_ref[0]
    k = k_ref[0]
    v = v_ref[0]
    s = _dot(q, k.T) * (HD ** -0.5)
    row = jax.lax.broadcasted_iota(jnp.int32, (T, T), 0)
    col = jax.lax.broadcasted_iota(jnp.int32, (T, T), 1)
    s = jnp.where(row >= col, s, -1e30)
    m = jnp.max(s, axis=-1, keepdims=True)
    p = jnp.exp(s - m)
    denom = jnp.sum(p, axis=-1, keepdims=True)
    p = p / denom
    o_ref[0] = _dot(p, v)


def _post_kernel(ao_ref, wo_ref, hs_ref, ln_ref, wg_ref,
                 h2_ref, h3_ref, g_ref):
    ao = jnp.concatenate([ao_ref[hh] for hh in range(NH)], axis=-1)
    attn_out = _dot(ao, wo_ref[...])
    h2 = attn_out + hs_ref[...]
    h3 = _rms(h2, ln_ref[...])
    h2_ref[...] = h2
    h3_ref[...] = h3

    logits = jax.lax.dot_general(
        h3, wg_ref[...], (((1,), (0,)), ((), ())),
        preferred_element_type=jnp.float32)
    scores = jax.nn.sigmoid(logits)
    idx = jax.lax.broadcasted_iota(jnp.int32, (T, E), 1)
    m1 = jnp.max(scores, axis=-1, keepdims=True)
    i1 = jnp.min(jnp.where(scores == m1, idx, E), axis=-1, keepdims=True)
    s2 = jnp.where(idx == i1, -jnp.inf, scores)
    m2 = jnp.max(s2, axis=-1, keepdims=True)
    i2 = jnp.min(jnp.where(s2 == m2, idx, E), axis=-1, keepdims=True)
    denom = m1 + m2 + 1e-20
    g = jnp.where(idx == i1, m1 / denom, 0.0) + \
        jnp.where(idx == i2, m2 / denom, 0.0)
    g_ref[...] = g


def _silu(x):
    return x * jax.nn.sigmoid(x)


def _moe_kernel(h3_ref, g_ref, h2_ref, eg_ref, eu_ref, ed_ref,
                sg_ref, su_ref, sd_ref, o_ref):
    e = pl.program_id(0)
    x = h3_ref[...]

    @pl.when(e == 0)
    def _():
        gate = _dot(x, sg_ref[...])
        up = _dot(x, su_ref[...])
        o_ref[...] = h2_ref[...] + _dot(_silu(gate) * up, sd_ref[...])

    gate = _dot(x, eg_ref[0])
    up = _dot(x, eu_ref[0])
    y = _dot(_silu(gate) * up, ed_ref[0])
    lane = jax.lax.broadcasted_iota(jnp.int32, (T, E), 1)
    w = jnp.sum(jnp.where(lane == e, g_ref[...], 0.0), axis=-1, keepdims=True)
    o_ref[...] += w * y


@functools.partial(jax.jit, static_argnames=())
def kernel(positions, hidden_states, Wq, Wk, Wv, Wo, q_norm_w, k_norm_w,
           in_ln_w, post_ln_w, Wg, We_gate, We_up, We_down, Ws_gate, Ws_up,
           Ws_down):
    pos2d = positions.reshape(T, 1)

    qkv = pl.pallas_call(
        _qkv_kernel,
        out_shape=(
            jax.ShapeDtypeStruct((NH, T, HD), jnp.float32),
            jax.ShapeDtypeStruct((NKV, T, HD), jnp.float32),
            jax.ShapeDtypeStruct((NKV, T, HD), jnp.float32),
        ),
    )
    q, k, v = qkv(pos2d, hidden_states, Wq, Wk, Wv, q_norm_w, k_norm_w,
                  in_ln_w)

    ao = pl.pallas_call(
        _attn_kernel,
        grid=(NH,),
        in_specs=[
            pl.BlockSpec((1, T, HD), lambda h: (h, 0, 0)),
            pl.BlockSpec((1, T, HD), lambda h: (h // REP, 0, 0)),
            pl.BlockSpec((1, T, HD), lambda h: (h // REP, 0, 0)),
        ],
        out_specs=pl.BlockSpec((1, T, HD), lambda h: (h, 0, 0)),
        out_shape=jax.ShapeDtypeStruct((NH, T, HD), jnp.float32),
    )(q, k, v)

    h2, h3, g = pl.pallas_call(
        _post_kernel,
        out_shape=(
            jax.ShapeDtypeStruct((T, H), jnp.float32),
            jax.ShapeDtypeStruct((T, H), jnp.float32),
            jax.ShapeDtypeStruct((T, E), jnp.float32),
        ),
    )(ao, Wo, hidden_states, post_ln_w, Wg)

    out = pl.pallas_call(
        _moe_kernel,
        grid=(E,),
        in_specs=[
            pl.BlockSpec((T, H), lambda e: (0, 0)),
            pl.BlockSpec((T, E), lambda e: (0, 0)),
            pl.BlockSpec((T, H), lambda e: (0, 0)),
            pl.BlockSpec((1, H, DFF), lambda e: (e, 0, 0)),
            pl.BlockSpec((1, H, DFF), lambda e: (e, 0, 0)),
            pl.BlockSpec((1, DFF, H), lambda e: (e, 0, 0)),
            pl.BlockSpec((H, DFF), lambda e: (0, 0)),
            pl.BlockSpec((H, DFF), lambda e: (0, 0)),
            pl.BlockSpec((DFF, H), lambda e: (0, 0)),
        ],
        out_specs=pl.BlockSpec((T, H), lambda e: (0, 0)),
        out_shape=jax.ShapeDtypeStruct((T, H), jnp.float32),
    )(h3, g, h2, We_gate, We_up, We_down, Ws_gate, Ws_up, Ws_down)

    return out


# speedup vs baseline: 2.0896x; 1.1474x over previous
import functools

import jax
import jax.numpy as jnp
from jax.experimental import pallas as pl
from jax.experimental.pallas import tpu as pltpu

H = 768
NH = 12
NKV = 4
HD = 64
E = 8
TOPK = 2
DFF = 512
T = 2048
THETA = 1000000.0
EPS = 1e-06
REP = NH // NKV


def _bf(x):
    return x.astype(jnp.bfloat16)


def _dot(a, b):
    return jax.lax.dot_general(
        _bf(a), _bf(b), (((1,), (0,)), ((), ())),
        preferred_element_type=jnp.float32)


def _rms(x, w):
    v = jnp.mean(jnp.square(x), axis=-1, keepdims=True)
    return x * jax.lax.rsqrt(v + EPS) * w


def _qkv_kernel(pos_ref, hs_ref, wq_ref, wk_ref, wv_ref, qn_ref, kn_ref,
                ln_ref, qo_ref, ko_ref, vo_ref):
    hs = hs_ref[...]
    h = _rms(hs, ln_ref[...])
    q = _dot(h, wq_ref[...])
    k = _dot(h, wk_ref[...])
    v = _dot(h, wv_ref[...])

    pos = pos_ref[...].astype(jnp.float32)
    half = HD // 2
    exponent = jax.lax.broadcasted_iota(jnp.int32, (1, half), 1).astype(
        jnp.float32) * (2.0 / HD)
    inv_freq = jnp.exp(-jnp.log(THETA) * exponent)
    ang = pos * inv_freq
    cos = jnp.cos(ang)
    sin = jnp.sin(ang)

    qw = qn_ref[...].reshape(1, HD)
    kw = kn_ref[...].reshape(1, HD)

    def norm_rope(x, w):
        xn = _rms(x, w)
        x1 = xn[:, :half]
        x2 = xn[:, half:]
        return jnp.concatenate([x1 * cos - x2 * sin, x2 * cos + x1 * sin],
                               axis=-1)

    for hh in range(NH):
        qo_ref[hh] = norm_rope(q[:, hh * HD:(hh + 1) * HD], qw)
    for hh in range(NKV):
        ko_ref[hh] = norm_rope(k[:, hh * HD:(hh + 1) * HD], kw)
        vo_ref[hh] = v[:, hh * HD:(hh + 1) * HD]


RB = 512
NRB = T // RB


def _attn_kernel(q_ref, k_ref, v_ref, o_ref):
    r = pl.program_id(1)
    q = _bf(q_ref[0] * (HD ** -0.5))

    def body(c, carry):
        acc, denom = carry
        k = _bf(k_ref[0, pl.ds(c * RB, RB), :])
        v = _bf(v_ref[0, pl.ds(c * RB, RB), :])
        s = jax.lax.dot_general(q, k, (((1,), (1,)), ((), ())),
                                preferred_element_type=jnp.float32)
        row = r * RB + jax.lax.broadcasted_iota(jnp.int32, (RB, RB), 0)
        col = c * RB + jax.lax.broadcasted_iota(jnp.int32, (RB, RB), 1)
        p = jnp.where(row >= col, jnp.exp(s), 0.0)
        acc = acc + jax.lax.dot_general(
            _bf(p), v, (((1,), (0,)), ((), ())),
            preferred_element_type=jnp.float32)
        denom = denom + jnp.sum(p, axis=-1, keepdims=True)
        return acc, denom

    acc = jnp.zeros((RB, HD), jnp.float32)
    denom = jnp.zeros((RB, 1), jnp.float32)
    acc, denom = jax.lax.fori_loop(0, r + 1, body, (acc, denom))
    o_ref[0] = acc / denom


def _post_kernel(ao_ref, wo_ref, hs_ref, ln_ref, wg_ref,
                 h2_ref, h3_ref, g_ref):
    ao = jnp.concatenate([ao_ref[hh] for hh in range(NH)], axis=-1)
    attn_out = _dot(ao, wo_ref[...])
    h2 = attn_out + hs_ref[...]
    h3 = _rms(h2, ln_ref[...])
    h2_ref[...] = h2
    h3_ref[...] = h3

    logits = jax.lax.dot_general(
        h3, wg_ref[...], (((1,), (0,)), ((), ())),
        preferred_element_type=jnp.float32)
    scores = jax.nn.sigmoid(logits)
    idx = jax.lax.broadcasted_iota(jnp.int32, (T, E), 1)
    m1 = jnp.max(scores, axis=-1, keepdims=True)
    i1 = jnp.min(jnp.where(scores == m1, idx, E), axis=-1, keepdims=True)
    s2 = jnp.where(idx == i1, -jnp.inf, scores)
    m2 = jnp.max(s2, axis=-1, keepdims=True)
    i2 = jnp.min(jnp.where(s2 == m2, idx, E), axis=-1, keepdims=True)
    denom = m1 + m2 + 1e-20
    g = jnp.where(idx == i1, m1 / denom, 0.0) + \
        jnp.where(idx == i2, m2 / denom, 0.0)
    g_ref[...] = g


def _silu(x):
    return x * jax.nn.sigmoid(x)


def _moe_kernel(h3_ref, g_ref, h2_ref, eg_ref, eu_ref, ed_ref,
                sg_ref, su_ref, sd_ref, o_ref):
    e = pl.program_id(0)
    x = h3_ref[...]

    @pl.when(e == 0)
    def _():
        gate = _dot(x, sg_ref[...])
        up = _dot(x, su_ref[...])
        o_ref[...] = h2_ref[...] + _dot(_silu(gate) * up, sd_ref[...])

    gate = _dot(x, eg_ref[0])
    up = _dot(x, eu_ref[0])
    y = _dot(_silu(gate) * up, ed_ref[0])
    lane = jax.lax.broadcasted_iota(jnp.int32, (T, E), 1)
    w = jnp.sum(jnp.where(lane == e, g_ref[...], 0.0), axis=-1, keepdims=True)
    o_ref[...] += w * y


@functools.partial(jax.jit, static_argnames=())
def kernel(positions, hidden_states, Wq, Wk, Wv, Wo, q_norm_w, k_norm_w,
           in_ln_w, post_ln_w, Wg, We_gate, We_up, We_down, Ws_gate, Ws_up,
           Ws_down):
    pos2d = positions.reshape(T, 1)

    qkv = pl.pallas_call(
        _qkv_kernel,
        out_shape=(
            jax.ShapeDtypeStruct((NH, T, HD), jnp.float32),
            jax.ShapeDtypeStruct((NKV, T, HD), jnp.float32),
            jax.ShapeDtypeStruct((NKV, T, HD), jnp.float32),
        ),
    )
    q, k, v = qkv(pos2d, hidden_states, Wq, Wk, Wv, q_norm_w, k_norm_w,
                  in_ln_w)

    ao = pl.pallas_call(
        _attn_kernel,
        grid=(NH, NRB),
        in_specs=[
            pl.BlockSpec((1, RB, HD), lambda h, r: (h, r, 0)),
            pl.BlockSpec((1, T, HD), lambda h, r: (h // REP, 0, 0)),
            pl.BlockSpec((1, T, HD), lambda h, r: (h // REP, 0, 0)),
        ],
        out_specs=pl.BlockSpec((1, RB, HD), lambda h, r: (h, r, 0)),
        out_shape=jax.ShapeDtypeStruct((NH, T, HD), jnp.float32),
    )(q, k, v)

    h2, h3, g = pl.pallas_call(
        _post_kernel,
        out_shape=(
            jax.ShapeDtypeStruct((T, H), jnp.float32),
            jax.ShapeDtypeStruct((T, H), jnp.float32),
            jax.ShapeDtypeStruct((T, E), jnp.float32),
        ),
    )(ao, Wo, hidden_states, post_ln_w, Wg)

    out = pl.pallas_call(
        _moe_kernel,
        grid=(E,),
        in_specs=[
            pl.BlockSpec((T, H), lambda e: (0, 0)),
            pl.BlockSpec((T, E), lambda e: (0, 0)),
            pl.BlockSpec((T, H), lambda e: (0, 0)),
            pl.BlockSpec((1, H, DFF), lambda e: (e, 0, 0)),
            pl.BlockSpec((1, H, DFF), lambda e: (e, 0, 0)),
            pl.BlockSpec((1, DFF, H), lambda e: (e, 0, 0)),
            pl.BlockSpec((H, DFF), lambda e: (0, 0)),
            pl.BlockSpec((H, DFF), lambda e: (0, 0)),
            pl.BlockSpec((DFF, H), lambda e: (0, 0)),
        ],
        out_specs=pl.BlockSpec((T, H), lambda e: (0, 0)),
        out_shape=jax.ShapeDtypeStruct((T, H), jnp.float32),
    )(h3, g, h2, We_gate, We_up, We_down, Ws_gate, Ws_up, Ws_down)

    return out


# speedup vs baseline: 2.2543x; 1.0788x over previous
import functools

import jax
import jax.numpy as jnp
from jax.experimental import pallas as pl
from jax.experimental.pallas import tpu as pltpu

H = 768
NH = 12
NKV = 4
HD = 64
E = 8
TOPK = 2
DFF = 512
T = 2048
THETA = 1000000.0
EPS = 1e-06
REP = NH // NKV


def _bf(x):
    return x.astype(jnp.bfloat16)


def _dot(a, b):
    return jax.lax.dot_general(
        _bf(a), _bf(b), (((1,), (0,)), ((), ())),
        preferred_element_type=jnp.float32)


def _rms(x, w):
    v = jnp.mean(jnp.square(x), axis=-1, keepdims=True)
    return x * jax.lax.rsqrt(v + EPS) * w


def _qkv_kernel(pos_ref, hs_ref, wq_ref, wk_ref, wv_ref, qn_ref, kn_ref,
                ln_ref, qo_ref, ko_ref, vo_ref):
    hs = hs_ref[...]
    h = _rms(hs, ln_ref[...])
    q = _dot(h, wq_ref[...])
    k = _dot(h, wk_ref[...])
    v = _dot(h, wv_ref[...])

    half = HD // 2
    pos = pos_ref[...].astype(jnp.float32)
    d128 = jax.lax.broadcasted_iota(jnp.int32, (1, 128), 1)
    inv128 = jnp.exp(-jnp.log(THETA) *
                     (d128 % half).astype(jnp.float32) / half)
    ang = pos * inv128
    cos128 = jnp.cos(ang)
    sin128 = jnp.sin(ang)

    def tile_lanes(x, w):
        return jnp.concatenate([x] * (w // x.shape[-1]), axis=-1)

    def norm_rope_full(x, w1, nheads):
        width = nheads * HD
        blk = (jax.lax.broadcasted_iota(jnp.int32, (width, nheads), 0) // HD
               == jax.lax.broadcasted_iota(jnp.int32, (width, nheads), 1)
               ).astype(jnp.float32)
        ms = jax.lax.dot_general(
            jnp.square(x), blk, (((1,), (0,)), ((), ())),
            preferred_element_type=jnp.float32) * (1.0 / HD)
        sf = jax.lax.rsqrt(ms + EPS)
        sfull = jax.lax.dot_general(
            sf, blk.T, (((1,), (0,)), ((), ())),
            preferred_element_type=jnp.float32)
        xn = x * sfull * tile_lanes(w1.reshape(1, HD), width)
        lane = jax.lax.broadcasted_iota(jnp.int32, (1, width), 1) % HD
        xl = pltpu.roll(xn, width - half, 1)
        xr = pltpu.roll(xn, half, 1)
        rot = jnp.where(lane < half, -xl, xr)
        cosf = tile_lanes(cos128, width)
        sinf = tile_lanes(sin128, width)
        return xn * cosf + rot * sinf

    qr = norm_rope_full(q, qn_ref[...], NH)
    kr = norm_rope_full(k, kn_ref[...], NKV)
    for hh in range(NH):
        qo_ref[hh] = qr[:, hh * HD:(hh + 1) * HD]
    for hh in range(NKV):
        ko_ref[hh] = kr[:, hh * HD:(hh + 1) * HD]
        vo_ref[hh] = v[:, hh * HD:(hh + 1) * HD]


RB = 512
NRB = T // RB


def _attn_kernel(q_ref, k_ref, v_ref, o_ref):
    r = pl.program_id(1)
    q = _bf(q_ref[0] * (HD ** -0.5))

    def body(c, carry):
        acc, denom = carry
        k = _bf(k_ref[0, pl.ds(c * RB, RB), :])
        v = _bf(v_ref[0, pl.ds(c * RB, RB), :])
        s = jax.lax.dot_general(q, k, (((1,), (1,)), ((), ())),
                                preferred_element_type=jnp.float32)
        row = r * RB + jax.lax.broadcasted_iota(jnp.int32, (RB, RB), 0)
        col = c * RB + jax.lax.broadcasted_iota(jnp.int32, (RB, RB), 1)
        p = jnp.where(row >= col, jnp.exp(s), 0.0)
        acc = acc + jax.lax.dot_general(
            _bf(p), v, (((1,), (0,)), ((), ())),
            preferred_element_type=jnp.float32)
        denom = denom + jnp.sum(p, axis=-1, keepdims=True)
        return acc, denom

    acc = jnp.zeros((RB, HD), jnp.float32)
    denom = jnp.zeros((RB, 1), jnp.float32)
    acc, denom = jax.lax.fori_loop(0, r + 1, body, (acc, denom))
    o_ref[0] = acc / denom


def _post_kernel(ao_ref, wo_ref, hs_ref, ln_ref, wg_ref,
                 h2_ref, h3_ref, g_ref):
    ao = jnp.concatenate([ao_ref[hh] for hh in range(NH)], axis=-1)
    attn_out = _dot(ao, wo_ref[...])
    h2 = attn_out + hs_ref[...]
    h3 = _rms(h2, ln_ref[...])
    h2_ref[...] = h2
    h3_ref[...] = h3

    logits = jax.lax.dot_general(
        h3, wg_ref[...], (((1,), (0,)), ((), ())),
        preferred_element_type=jnp.float32)
    scores = jax.nn.sigmoid(logits)
    idx = jax.lax.broadcasted_iota(jnp.int32, (T, E), 1)
    m1 = jnp.max(scores, axis=-1, keepdims=True)
    i1 = jnp.min(jnp.where(scores == m1, idx, E), axis=-1, keepdims=True)
    s2 = jnp.where(idx == i1, -jnp.inf, scores)
    m2 = jnp.max(s2, axis=-1, keepdims=True)
    i2 = jnp.min(jnp.where(s2 == m2, idx, E), axis=-1, keepdims=True)
    denom = m1 + m2 + 1e-20
    g = jnp.where(idx == i1, m1 / denom, 0.0) + \
        jnp.where(idx == i2, m2 / denom, 0.0)
    g_ref[...] = g


def _silu(x):
    return x * jax.nn.sigmoid(x)


def _moe_kernel(h3_ref, g_ref, h2_ref, eg_ref, eu_ref, ed_ref,
                sg_ref, su_ref, sd_ref, o_ref):
    e = pl.program_id(0)
    x = _bf(h3_ref[...])

    def mlp(g_w, u_w, d_w):
        gate = jax.lax.dot_general(x, _bf(g_w), (((1,), (0,)), ((), ())),
                                   preferred_element_type=jnp.float32)
        up = jax.lax.dot_general(x, _bf(u_w), (((1,), (0,)), ((), ())),
                                 preferred_element_type=jnp.float32)
        return jax.lax.dot_general(_bf(_silu(gate) * up), _bf(d_w),
                                   (((1,), (0,)), ((), ())),
                                   preferred_element_type=jnp.float32)

    @pl.when(e == 0)
    def _():
        o_ref[...] = h2_ref[...] + mlp(sg_ref[...], su_ref[...], sd_ref[...])

    y = mlp(eg_ref[0], eu_ref[0], ed_ref[0])
    lane = jax.lax.broadcasted_iota(jnp.int32, (T, E), 1)
    w = jnp.sum(jnp.where(lane == e, g_ref[...], 0.0), axis=-1, keepdims=True)
    o_ref[...] += w * y


@functools.partial(jax.jit, static_argnames=())
def kernel(positions, hidden_states, Wq, Wk, Wv, Wo, q_norm_w, k_norm_w,
           in_ln_w, post_ln_w, Wg, We_gate, We_up, We_down, Ws_gate, Ws_up,
           Ws_down):
    pos2d = positions.reshape(T, 1)

    qkv = pl.pallas_call(
        _qkv_kernel,
        out_shape=(
            jax.ShapeDtypeStruct((NH, T, HD), jnp.float32),
            jax.ShapeDtypeStruct((NKV, T, HD), jnp.float32),
            jax.ShapeDtypeStruct((NKV, T, HD), jnp.float32),
        ),
    )
    q, k, v = qkv(pos2d, hidden_states, Wq, Wk, Wv, q_norm_w, k_norm_w,
                  in_ln_w)

    ao = pl.pallas_call(
        _attn_kernel,
        grid=(NH, NRB),
        in_specs=[
            pl.BlockSpec((1, RB, HD), lambda h, r: (h, r, 0)),
            pl.BlockSpec((1, T, HD), lambda h, r: (h // REP, 0, 0)),
            pl.BlockSpec((1, T, HD), lambda h, r: (h // REP, 0, 0)),
        ],
        out_specs=pl.BlockSpec((1, RB, HD), lambda h, r: (h, r, 0)),
        out_shape=jax.ShapeDtypeStruct((NH, T, HD), jnp.float32),
    )(q, k, v)

    h2, h3, g = pl.pallas_call(
        _post_kernel,
        out_shape=(
            jax.ShapeDtypeStruct((T, H), jnp.float32),
            jax.ShapeDtypeStruct((T, H), jnp.float32),
            jax.ShapeDtypeStruct((T, E), jnp.float32),
        ),
    )(ao, Wo, hidden_states, post_ln_w, Wg)

    out = pl.pallas_call(
        _moe_kernel,
        grid=(E,),
        in_specs=[
            pl.BlockSpec((T, H), lambda e: (0, 0)),
            pl.BlockSpec((T, E), lambda e: (0, 0)),
            pl.BlockSpec((T, H), lambda e: (0, 0)),
            pl.BlockSpec((1, H, DFF), lambda e: (e, 0, 0)),
            pl.BlockSpec((1, H, DFF), lambda e: (e, 0, 0)),
            pl.BlockSpec((1, DFF, H), lambda e: (e, 0, 0)),
            pl.BlockSpec((H, DFF), lambda e: (0, 0)),
            pl.BlockSpec((H, DFF), lambda e: (0, 0)),
            pl.BlockSpec((DFF, H), lambda e: (0, 0)),
        ],
        out_specs=pl.BlockSpec((T, H), lambda e: (0, 0)),
        out_shape=jax.ShapeDtypeStruct((T, H), jnp.float32),
    )(h3, g, h2, We_gate, We_up, We_down, Ws_gate, Ws_up, Ws_down)

    return out
